# baseline (speedup 1.0000x reference)
import functools

import jax
import jax.numpy as jnp
from jax import lax
from jax.experimental import pallas as pl
from jax.experimental.pallas import tpu as pltpu
from jax.experimental.pallas import tpu_sc as plsc

_NC = 2
_NS = 16
_NW = _NC * _NS
_LANES = 16


def _positional_encoding(static_len: int, dims: int) -> jnp.ndarray:
    pos = jnp.arange(static_len, dtype=jnp.float32)[:, None]
    i = jnp.arange(dims, dtype=jnp.float32)[None, :]
    angle = pos / jnp.power(10000.0, 2.0 * i / dims)
    even = jnp.sin(angle)
    odd = jnp.cos(angle)
    col = jnp.arange(dims)[None, :]
    pe = jnp.where(col % 2 == 0, even, odd)
    pe = pe.at[0].set(0.0)
    return pe


@functools.partial(jax.jit, static_argnames=("vocab", "dims"))
def _sc_transpose(table_t, *, vocab, dims):
    lanes = 128
    n_blocks = (vocab + lanes - 1) // lanes
    mesh = plsc.VectorSubcoreMesh(
        core_axis_name="c", subcore_axis_name="s", num_cores=_NC, num_subcores=_NS
    )

    kw = 2
    wcols = kw * lanes
    pad = wcols + 1
    n_pairs = n_blocks // kw
    dsub = dims // 8

    @functools.partial(
        pl.kernel,
        out_type=jax.ShapeDtypeStruct((vocab // 2, 2 * dims), jnp.float32),
        mesh=mesh,
        scratch_types=[
            [pltpu.VMEM((dims, pad), jnp.float32) for _ in range(2)],
            [pltpu.VMEM((kw * dims, lanes), jnp.float32) for _ in range(2)],
            [pltpu.SemaphoreType.DMA for _ in range(2)],
            [pltpu.SemaphoreType.DMA for _ in range(2)],
        ],
        compiler_params=pltpu.CompilerParams(needs_layout_passes=False),
    )
    def body(tt_hbm, out_hbm, srcs, dsts, rsem, wsem):
        wid = lax.axis_index("s") * _NC + lax.axis_index("c")

        def read_descs(q, p):
            return [
                pltpu.make_async_copy(
                    tt_hbm.at[pl.ds(dg * 8, 8), pl.ds(q * wcols, wcols)],
                    srcs[p].at[pl.ds(dg * 8, 8), pl.ds(0, wcols)],
                    rsem[p],
                )
                for dg in range(dsub)
            ]

        def wr_desc(q, p):
            return pltpu.make_async_copy(
                dsts[p], out_hbm.at[pl.ds(q * (wcols // 2), kw * dims)], wsem[p]
            )

        def transpose_block(p, ncols):
            src_v, dst_v = srcs[p], dsts[p]

            @pl.loop(0, ncols, unroll=4)
            def _col_loop(v):
                half = (v & 1) * dims
                row = v >> 1
                v_idx = jnp.zeros((_LANES,), jnp.int32) + v
                for dg in range(dims // _LANES):
                    d_idx = lax.iota(jnp.int32, _LANES) + dg * _LANES
                    col = plsc.load_gather(src_v, [d_idx, v_idx])
                    dst_v[row, pl.ds(half + dg * _LANES, _LANES)] = col

        def do_step(i, q, p):
            nxt = q + _NW

            @pl.when(nxt < n_pairs)
            def _():
                @pl.when(i >= 1)
                def _():
                    wr_desc(0, 1 - p).wait()

                for d in read_descs(nxt, 1 - p):
                    d.start()

            for d in read_descs(q, p):
                d.wait()
            transpose_block(p, wcols)
            wr_desc(q, p).start()

        for d in read_descs(wid, 0):
            d.start()

        n_iter = (n_pairs + _NW - 1) // _NW

        @pl.loop(0, n_iter // 2)
        def _blk_loop(g):
            for sub in range(2):
                i = 2 * g + sub
                q = wid + i * _NW
                @pl.when(q < n_pairs)
                def _(i=i, q=q, sub=sub):
                    do_step(i, q, sub)

        if n_iter % 2:
            i = n_iter - 1
            q = wid + i * _NW

            @pl.when(q < n_pairs)
            def _():
                do_step(i, q, (n_iter - 1) % 2)

        wr_desc(0, 0).wait()
        wr_desc(0, 1).wait()

    return body(table_t)


@functools.partial(jax.jit, static_argnames=("n_rows", "dims", "chunk", "n_chunks"))
def _sc_embed(table2, idx2, off2, pe_flat, *, n_rows, dims, chunk, n_chunks):
    rows_per_w = n_rows // _NW
    half = chunk // 2
    pe_rows = 4 * 50
    mesh = plsc.VectorSubcoreMesh(
        core_axis_name="c", subcore_axis_name="s", num_cores=_NC, num_subcores=_NS
    )
    nbuf = 3

    @functools.partial(
        pl.kernel,
        out_type=jax.ShapeDtypeStruct((n_rows // 2, 2 * dims), jnp.float32),
        mesh=mesh,
        scratch_types=[
            pltpu.VMEM((rows_per_w,), jnp.int32),
            pltpu.VMEM((rows_per_w + _LANES,), jnp.int32),
            pltpu.VMEM((pe_rows * dims,), jnp.float32),
            [pltpu.VMEM((chunk, 2 * dims), jnp.float32) for _ in range(nbuf)],
            [pltpu.SemaphoreType.DMA for _ in range(nbuf)],
            [pltpu.SemaphoreType.DMA for _ in range(nbuf)],
        ],
    )
    def body(
        table_hbm, idx_hbm, off_hbm, pe_hbm, out_hbm,
        idx_v, off_v, pe_v, rows, gsem, osem,
    ):
        wid = lax.axis_index("s") * _NC + lax.axis_index("c")
        base = wid * rows_per_w
        obase = wid * (rows_per_w // 2)
        pltpu.sync_copy(idx_hbm.at[pl.ds(base, rows_per_w)], idx_v)
        pltpu.sync_copy(
            off_hbm.at[pl.ds(base, rows_per_w)], off_v.at[pl.ds(0, rows_per_w)]
        )
        pltpu.sync_copy(pe_hbm, pe_v)

        def gather_desc(c, b):
            return pltpu.make_async_copy(
                table_hbm.at[idx_v.at[pl.ds(c * chunk, chunk)]], rows[b], gsem[b]
            )

        def out_desc(c, b):
            return pltpu.make_async_copy(
                rows[b].at[pl.ds(0, half)],
                out_hbm.at[pl.ds(obase + c * half, half)],
                osem[b],
            )

        def compute(c, b):
            rows_v = rows[b]
            phase = lax.rem(c, 5) * 10

            @pl.loop(0, half, unroll=2)
            def _pair_loop(k, rows_v=rows_v, phase=phase, c=c):
                offv = off_v[pl.ds(c * chunk + 2 * k, _LANES)]
                pe_base = (phase + 2 * k) * dims
                for t in range(2):
                    off = offv[t]
                    for j in range(dims // _LANES):
                        src = rows_v[2 * k + t, pl.ds(off + j * _LANES, _LANES)]
                        pv = pe_v[pl.ds(pe_base + t * dims + j * _LANES, _LANES)]
                        rows_v[k, pl.ds(t * dims + j * _LANES, _LANES)] = src + pv

        def step(c, b, drain, prefetch):
            gather_desc(c, b).wait()
            compute(c, b)
            out_desc(c, b).start()
            if prefetch:
                pb = (b + nbuf - 1) % nbuf
                if drain:
                    out_desc(c - 1, pb).wait()
                gather_desc(c + nbuf - 1, pb).start()

        for c in range(nbuf - 1):
            gather_desc(c, c % nbuf).start()

        for b in range(nbuf):
            step(b, b, drain=(b >= 1), prefetch=True)

        n_blocks = n_chunks // nbuf
        last_full = n_blocks - 1

        @pl.loop(1, last_full)
        def _block_loop(g):
            for b in range(nbuf):
                step(g * nbuf + b, b, drain=True, prefetch=True)

        for c in range(last_full * nbuf, n_chunks):
            step(c, c % nbuf, drain=True, prefetch=(c + nbuf - 1 < n_chunks))

        for c in range(n_chunks - nbuf, n_chunks):
            out_desc(c, c % nbuf).wait()

    return body(table2, idx2, off2, pe_flat)


def kernel(x, cutoff_max_sen_len, vocab_size, table):
    batch, seq = x.shape
    _, dims = table.shape
    n_rows = batch * seq

    chunk = 160
    assert n_rows % (_NW * chunk) == 0
    n_chunks = n_rows // (_NW * chunk)

    pe = _positional_encoding(seq, dims)
    pe_flat = jnp.tile(pe, (4, 1)).reshape(-1)

    table2 = _sc_transpose(table.T, vocab=table.shape[0], dims=dims)
    n_tail = table.shape[0] % 256
    tail2 = table[-n_tail:].reshape(n_tail // 2, 2 * dims)
    table2 = table2.at[-(n_tail // 2):].set(tail2)
    flat = x.reshape(-1)
    idx2 = flat >> 1
    off2 = (flat & 1) * dims

    out2 = _sc_embed(
        table2, idx2, off2, pe_flat,
        n_rows=n_rows, dims=dims, chunk=chunk, n_chunks=n_chunks,
    )
    return out2.reshape(batch, seq, dims)


# speedup vs baseline: 1.5906x; 1.5906x over previous
import functools

import jax
import jax.numpy as jnp
from jax import lax
from jax.experimental import pallas as pl
from jax.experimental.pallas import tpu as pltpu
from jax.experimental.pallas import tpu_sc as plsc

_NC = 2
_NS = 16
_NW = _NC * _NS
_LANES = 16


def _positional_encoding(static_len: int, dims: int) -> jnp.ndarray:
    pos = jnp.arange(static_len, dtype=jnp.float32)[:, None]
    i = jnp.arange(dims, dtype=jnp.float32)[None, :]
    angle = pos / jnp.power(10000.0, 2.0 * i / dims)
    even = jnp.sin(angle)
    odd = jnp.cos(angle)
    col = jnp.arange(dims)[None, :]
    pe = jnp.where(col % 2 == 0, even, odd)
    pe = pe.at[0].set(0.0)
    return pe


@functools.partial(jax.jit, static_argnames=("vocab", "dims"))
def _sc_transpose(table_t, *, vocab, dims):
    lanes = 128
    n_blocks = (vocab + lanes - 1) // lanes
    mesh = plsc.VectorSubcoreMesh(
        core_axis_name="c", subcore_axis_name="s", num_cores=_NC, num_subcores=_NS
    )

    kw = 2
    wcols = kw * lanes
    pad = wcols + 1
    n_pairs = n_blocks // kw
    dsub = dims // 8

    @functools.partial(
        pl.kernel,
        out_type=jax.ShapeDtypeStruct((vocab // 2, 2 * dims), jnp.float32),
        mesh=mesh,
        scratch_types=[
            [pltpu.VMEM((dims, pad), jnp.float32) for _ in range(2)],
            [pltpu.VMEM((kw * dims, lanes), jnp.float32) for _ in range(2)],
            [pltpu.SemaphoreType.DMA for _ in range(2)],
            [pltpu.SemaphoreType.DMA for _ in range(2)],
        ],
        compiler_params=pltpu.CompilerParams(needs_layout_passes=False),
    )
    def body(tt_hbm, out_hbm, srcs, dsts, rsem, wsem):
        wid = lax.axis_index("s") * _NC + lax.axis_index("c")

        def read_descs(q, p):
            return [
                pltpu.make_async_copy(
                    tt_hbm.at[pl.ds(dg * 8, 8), pl.ds(q * wcols, wcols)],
                    srcs[p].at[pl.ds(dg * 8, 8), pl.ds(0, wcols)],
                    rsem[p],
                )
                for dg in range(dsub)
            ]

        def wr_desc(q, p):
            return pltpu.make_async_copy(
                dsts[p], out_hbm.at[pl.ds(q * (wcols // 2), kw * dims)], wsem[p]
            )

        d_idx_vecs = [
            lax.iota(jnp.int32, _LANES) + dg * _LANES for dg in range(dims // _LANES)
        ]
        zeros16 = jnp.zeros((_LANES,), jnp.int32)

        def transpose_block(p, ncols):
            src_v, dst_v = srcs[p], dsts[p]

            @plsc.parallel_loop(0, ncols, 1, unroll=2)
            def _col_loop(v):
                half = (v & 1) * dims
                row = v >> 1
                v_idx = zeros16 + v
                for dg in range(dims // _LANES):
                    col = plsc.load_gather(src_v, [d_idx_vecs[dg], v_idx])
                    dst_v[row, pl.ds(half + dg * _LANES, _LANES)] = col

        def do_step(i, q, p):
            nxt = q + _NW

            @pl.when(nxt < n_pairs)
            def _():
                @pl.when(i >= 1)
                def _():
                    wr_desc(0, 1 - p).wait()

                for d in read_descs(nxt, 1 - p):
                    d.start()

            for d in read_descs(q, p):
                d.wait()
            transpose_block(p, wcols)
            wr_desc(q, p).start()

        for d in read_descs(wid, 0):
            d.start()

        n_iter = (n_pairs + _NW - 1) // _NW

        @pl.loop(0, n_iter // 2)
        def _blk_loop(g):
            for sub in range(2):
                i = 2 * g + sub
                q = wid + i * _NW
                @pl.when(q < n_pairs)
                def _(i=i, q=q, sub=sub):
                    do_step(i, q, sub)

        if n_iter % 2:
            i = n_iter - 1
            q = wid + i * _NW

            @pl.when(q < n_pairs)
            def _():
                do_step(i, q, (n_iter - 1) % 2)

        wr_desc(0, 0).wait()
        wr_desc(0, 1).wait()

    return body(table_t)


@functools.partial(jax.jit, static_argnames=("n_rows", "dims", "chunk", "n_chunks"))
def _sc_embed(table2, idx2, off2, pe_flat, *, n_rows, dims, chunk, n_chunks):
    rows_per_w = n_rows // _NW
    half = chunk // 2
    pe_rows = 4 * 50
    mesh = plsc.VectorSubcoreMesh(
        core_axis_name="c", subcore_axis_name="s", num_cores=_NC, num_subcores=_NS
    )
    nbuf = 3

    @functools.partial(
        pl.kernel,
        out_type=jax.ShapeDtypeStruct((n_rows // 2, 2 * dims), jnp.float32),
        mesh=mesh,
        scratch_types=[
            pltpu.VMEM((rows_per_w,), jnp.int32),
            pltpu.VMEM((rows_per_w + _LANES,), jnp.int32),
            pltpu.VMEM((pe_rows * dims,), jnp.float32),
            [pltpu.VMEM((chunk, 2 * dims), jnp.float32) for _ in range(nbuf)],
            [pltpu.SemaphoreType.DMA for _ in range(nbuf)],
            [pltpu.SemaphoreType.DMA for _ in range(nbuf)],
        ],
    )
    def body(
        table_hbm, idx_hbm, off_hbm, pe_hbm, out_hbm,
        idx_v, off_v, pe_v, rows, gsem, osem,
    ):
        wid = lax.axis_index("s") * _NC + lax.axis_index("c")
        base = wid * rows_per_w
        obase = wid * (rows_per_w // 2)
        pltpu.sync_copy(idx_hbm.at[pl.ds(base, rows_per_w)], idx_v)
        pltpu.sync_copy(
            off_hbm.at[pl.ds(base, rows_per_w)], off_v.at[pl.ds(0, rows_per_w)]
        )
        pltpu.sync_copy(pe_hbm, pe_v)

        def gather_desc(c, b):
            return pltpu.make_async_copy(
                table_hbm.at[idx_v.at[pl.ds(c * chunk, chunk)]], rows[b], gsem[b]
            )

        def out_desc(c, b):
            return pltpu.make_async_copy(
                rows[b].at[pl.ds(0, half)],
                out_hbm.at[pl.ds(obase + c * half, half)],
                osem[b],
            )

        def compute(c, b):
            rows_v = rows[b]
            phase = lax.rem(c, 5) * 10

            @pl.loop(0, half, unroll=2)
            def _pair_loop(k, rows_v=rows_v, phase=phase, c=c):
                offv = off_v[pl.ds(c * chunk + 2 * k, _LANES)]
                pe_base = (phase + 2 * k) * dims
                for t in range(2):
                    off = offv[t]
                    for j in range(dims // _LANES):
                        src = rows_v[2 * k + t, pl.ds(off + j * _LANES, _LANES)]
                        pv = pe_v[pl.ds(pe_base + t * dims + j * _LANES, _LANES)]
                        rows_v[k, pl.ds(t * dims + j * _LANES, _LANES)] = src + pv

        def step(c, b, drain, prefetch):
            gather_desc(c, b).wait()
            compute(c, b)
            out_desc(c, b).start()
            if prefetch:
                pb = (b + nbuf - 1) % nbuf
                if drain:
                    out_desc(c - 1, pb).wait()
                gather_desc(c + nbuf - 1, pb).start()

        for c in range(nbuf - 1):
            gather_desc(c, c % nbuf).start()

        for b in range(nbuf):
            step(b, b, drain=(b >= 1), prefetch=True)

        n_blocks = n_chunks // nbuf
        last_full = n_blocks - 1

        @pl.loop(1, last_full)
        def _block_loop(g):
            for b in range(nbuf):
                step(g * nbuf + b, b, drain=True, prefetch=True)

        for c in range(last_full * nbuf, n_chunks):
            step(c, c % nbuf, drain=True, prefetch=(c + nbuf - 1 < n_chunks))

        for c in range(n_chunks - nbuf, n_chunks):
            out_desc(c, c % nbuf).wait()

    return body(table2, idx2, off2, pe_flat)


def kernel(x, cutoff_max_sen_len, vocab_size, table):
    batch, seq = x.shape
    _, dims = table.shape
    n_rows = batch * seq

    chunk = 160
    assert n_rows % (_NW * chunk) == 0
    n_chunks = n_rows // (_NW * chunk)

    pe = _positional_encoding(seq, dims)
    pe_flat = jnp.tile(pe, (4, 1)).reshape(-1)

    table2 = _sc_transpose(table.T, vocab=table.shape[0], dims=dims)
    n_tail = table.shape[0] % 256
    tail2 = table[-n_tail:].reshape(n_tail // 2, 2 * dims)
    table2 = table2.at[-(n_tail // 2):].set(tail2)
    flat = x.reshape(-1)
    idx2 = flat >> 1
    off2 = (flat & 1) * dims

    out2 = _sc_embed(
        table2, idx2, off2, pe_flat,
        n_rows=n_rows, dims=dims, chunk=chunk, n_chunks=n_chunks,
    )
    return out2.reshape(batch, seq, dims)


# speedup vs baseline: 2.3410x; 1.4717x over previous
import functools

import jax
import jax.numpy as jnp
from jax import lax
from jax.experimental import pallas as pl
from jax.experimental.pallas import tpu as pltpu
from jax.experimental.pallas import tpu_sc as plsc

_NC = 2
_NS = 16
_NW = _NC * _NS
_LANES = 16


def _positional_encoding(static_len: int, dims: int) -> jnp.ndarray:
    pos = jnp.arange(static_len, dtype=jnp.float32)[:, None]
    i = jnp.arange(dims, dtype=jnp.float32)[None, :]
    angle = pos / jnp.power(10000.0, 2.0 * i / dims)
    even = jnp.sin(angle)
    odd = jnp.cos(angle)
    col = jnp.arange(dims)[None, :]
    pe = jnp.where(col % 2 == 0, even, odd)
    pe = pe.at[0].set(0.0)
    return pe


@functools.partial(jax.jit, static_argnames=("n_rows", "dims", "chunk", "n_chunks"))
def _sc_embed(table, idx3, pe_tile, *, n_rows, dims, chunk, n_chunks):
    rows_per_w = n_rows // _NW
    mesh = plsc.VectorSubcoreMesh(
        core_axis_name="c", subcore_axis_name="s", num_cores=_NC, num_subcores=_NS
    )

    @functools.partial(
        pl.kernel,
        out_type=jax.ShapeDtypeStruct((n_rows, dims), jnp.float32),
        mesh=mesh,
        scratch_types=[
            pltpu.VMEM((n_chunks * chunk,), jnp.int32),
            pltpu.VMEM((chunk, dims), jnp.float32),
            pltpu.VMEM((chunk, dims), jnp.float32),
            pltpu.SemaphoreType.DMA,
        ],
        compiler_params=pltpu.CompilerParams(use_tc_tiling_on_sc=False),
    )
    def body(table_hbm, idx_hbm, pe_hbm, out_hbm, idx_v, pe_v, rows_v, sem):
        wid = lax.axis_index("s") * _NC + lax.axis_index("c")
        base = wid * rows_per_w
        pltpu.sync_copy(idx_hbm.at[wid], idx_v)
        pltpu.sync_copy(pe_hbm, pe_v)

        @pl.loop(0, n_chunks)
        def _chunk_loop(c):
            pltpu.async_copy(
                table_hbm.at[idx_v.at[pl.ds(c * chunk, chunk)]], rows_v, sem
            ).wait()

            @pl.loop(0, chunk)
            def _row_loop(r):
                for j in range(dims // _LANES):
                    sl = pl.ds(j * _LANES, _LANES)
                    rows_v[r, sl] = rows_v[r, sl] + pe_v[r, sl]

            pltpu.sync_copy(rows_v, out_hbm.at[pl.ds(base + c * chunk, chunk)])

    return body(table, idx3, pe_tile)


def kernel(x, cutoff_max_sen_len, vocab_size, table):
    batch, seq = x.shape
    _, dims = table.shape
    n_rows = batch * seq

    chunk = 400
    assert chunk % seq == 0 and n_rows % (_NW * chunk) == 0
    n_chunks = n_rows // (_NW * chunk)

    pe = _positional_encoding(seq, dims)
    pe_tile = jnp.tile(pe, (chunk // seq, 1))
    idx3 = x.reshape(_NW, n_chunks * chunk)

    out = _sc_embed(
        table, idx3, pe_tile, n_rows=n_rows, dims=dims, chunk=chunk, n_chunks=n_chunks
    )
    return out.reshape(batch, seq, dims)
